# baseline (speedup 1.0000x reference)
import jax
import jax.numpy as jnp
from jax import lax
from jax.experimental import pallas as pl
from jax.experimental.pallas import tpu as pltpu
from jax.experimental.pallas import tpu_sc as plsc

ROWS, COLS, K = 16384, 200, 4
N_IDX = ROWS * COLS
OUT_W = COLS * K

_info = plsc.get_sparse_core_info()
NC, NS, L = _info.num_cores, _info.num_subcores, _info.num_lanes
NW = NC * NS

ROWS_PER_W = ROWS // NW
CHUNK_R = 8
CHUNK_IDX = CHUNK_R * COLS
N_CHUNKS = ROWS_PER_W // CHUNK_R
VECS_PER_ROW = OUT_W // L
N_VECS = CHUNK_R * VECS_PER_ROW


def _sc_body(seq_hbm, basis_hbm, out_hbm, idx_v, out_v, basis_v,
             sem_in, sem_out):
    wid = lax.axis_index("s") * NC + lax.axis_index("c")
    row0 = wid * ROWS_PER_W
    idx0 = wid * ROWS_PER_W * COLS

    pltpu.make_async_copy(basis_hbm, basis_v, sem_in).start()
    pltpu.make_async_copy(basis_hbm, basis_v, sem_in).wait()

    iota = lax.iota(jnp.int32, L)
    div4 = lax.shift_right_logical(iota, 2)
    mod4 = lax.bitwise_and(iota, 3)

    def chunk_body(g, _):
        pltpu.make_async_copy(
            seq_hbm.at[pl.ds(idx0 + g * CHUNK_IDX, CHUNK_IDX)], idx_v,
            sem_in).start()
        pltpu.make_async_copy(
            seq_hbm.at[pl.ds(idx0 + g * CHUNK_IDX, CHUNK_IDX)], idx_v,
            sem_in).wait()

        @plsc.parallel_loop(0, N_VECS, 1, unroll=8)
        def vec_body(q):
            ri = q // VECS_PER_ROW
            s = q - ri * VECS_PER_ROW
            jv = lax.broadcast(ri * COLS + s * K, (L,)) + div4
            idx16 = plsc.load_gather(idx_v, [jv])
            addr = lax.shift_left(
                idx16, lax.broadcast(jnp.int32(2), (L,))) + mod4
            out_v[ri, pl.ds(s * L, L)] = plsc.load_gather(basis_v, [addr])

        pltpu.make_async_copy(
            out_v, out_hbm.at[pl.ds(row0 + g * CHUNK_R, CHUNK_R), :],
            sem_out).start()
        pltpu.make_async_copy(
            out_v, out_hbm.at[pl.ds(row0 + g * CHUNK_R, CHUNK_R), :],
            sem_out).wait()
        return ()

    lax.fori_loop(0, N_CHUNKS, chunk_body, ())


@jax.jit
def _su2_lookup(seq_flat, basis_flat):
    mesh = plsc.VectorSubcoreMesh(core_axis_name="c", subcore_axis_name="s")
    out2 = pl.kernel(
        _sc_body,
        mesh=mesh,
        compiler_params=pltpu.CompilerParams(needs_layout_passes=False),
        out_type=jax.ShapeDtypeStruct((ROWS, OUT_W), jnp.float32),
        scratch_types=[
            pltpu.VMEM((CHUNK_IDX,), jnp.int32),
            pltpu.VMEM((CHUNK_R, OUT_W), jnp.float32),
            pltpu.VMEM((K * K,), jnp.float32),
            pltpu.SemaphoreType.DMA,
            pltpu.SemaphoreType.DMA,
        ],
    )(seq_flat, basis_flat)
    return out2.reshape(ROWS, COLS, K)


def kernel(sequence_indices, basis):
    seq_flat = sequence_indices.reshape(-1).astype(jnp.int32)
    return _su2_lookup(seq_flat, basis.reshape(-1))


# speedup vs baseline: 1.6959x; 1.6959x over previous
import jax
import jax.numpy as jnp
from jax import lax
from jax.experimental import pallas as pl
from jax.experimental.pallas import tpu as pltpu
from jax.experimental.pallas import tpu_sc as plsc

ROWS, COLS, K = 16384, 200, 4
OUT_W = COLS * K

_info = plsc.get_sparse_core_info()
NC, NS, L = _info.num_cores, _info.num_subcores, _info.num_lanes
NW = NC * NS

ROWS_PER_W = ROWS // NW
CHUNK_R = 16
N_CHUNKS = ROWS_PER_W // CHUNK_R
VECS_PER_ROW = OUT_W // L


def _sc_body(seq_hbm, basis_hbm, out_hbm, idx0_v, idx1_v, out0_v, out1_v,
             basis_v, sin0, sin1, sout0, sout1):
    wid = lax.axis_index("s") * NC + lax.axis_index("c")
    row0 = wid * ROWS_PER_W

    pltpu.make_async_copy(basis_hbm, basis_v, sin0).start()
    pltpu.make_async_copy(basis_hbm, basis_v, sin0).wait()

    iota = lax.iota(jnp.int32, L)
    div4 = lax.shift_right_logical(iota, 2)
    mod4 = lax.bitwise_and(iota, 3)
    two = lax.broadcast(jnp.int32(2), (L,))

    def seq_band(g):
        return seq_hbm.at[pl.ds(row0 + g * CHUNK_R, CHUNK_R), :]

    def out_band(g):
        return out_hbm.at[pl.ds(row0 + g * CHUNK_R, CHUNK_R), :]

    bufs = ((idx0_v, out0_v, sin0, sout0), (idx1_v, out1_v, sin1, sout1))

    pltpu.make_async_copy(seq_band(0), idx0_v, sin0).start()

    def pair_body(g2, _):
        for b, (iv, ov, si, so) in enumerate(bufs):
            g = g2 * 2 + b
            pltpu.make_async_copy(seq_band(g), iv, si).wait()

            @pl.when(g < N_CHUNKS - 1)
            def _():
                nxt = bufs[1 - b]
                pltpu.make_async_copy(seq_band(g + 1), nxt[0], nxt[2]).start()

            @pl.when(g >= 2)
            def _():
                pltpu.make_async_copy(ov, out_band(g), so).wait()

            for ri in range(CHUNK_R):
                riv = lax.broadcast(jnp.int32(ri), (L,))

                @plsc.parallel_loop(0, VECS_PER_ROW, 1, unroll=10)
                def vec_body(s):
                    jv = lax.broadcast(s * K, (L,)) + div4
                    idx16 = plsc.load_gather(iv, [riv, jv])
                    addr = lax.shift_left(idx16, two) + mod4
                    ov[ri, pl.ds(s * L, L)] = plsc.load_gather(basis_v, [addr])

            pltpu.make_async_copy(ov, out_band(g), so).start()
        return ()

    lax.fori_loop(0, N_CHUNKS // 2, pair_body, ())

    for b, (iv, ov, si, so) in enumerate(bufs):
        pltpu.make_async_copy(ov, out_band(N_CHUNKS - 2 + b), so).wait()


@jax.jit
def _su2_lookup(seq, basis_flat):
    mesh = plsc.VectorSubcoreMesh(core_axis_name="c", subcore_axis_name="s")
    out2 = pl.kernel(
        _sc_body,
        mesh=mesh,
        compiler_params=pltpu.CompilerParams(needs_layout_passes=False),
        out_type=jax.ShapeDtypeStruct((ROWS, OUT_W), jnp.float32),
        scratch_types=[
            pltpu.VMEM((CHUNK_R, COLS), jnp.int32),
            pltpu.VMEM((CHUNK_R, COLS), jnp.int32),
            pltpu.VMEM((CHUNK_R, OUT_W), jnp.float32),
            pltpu.VMEM((CHUNK_R, OUT_W), jnp.float32),
            pltpu.VMEM((K * K,), jnp.float32),
            pltpu.SemaphoreType.DMA,
            pltpu.SemaphoreType.DMA,
            pltpu.SemaphoreType.DMA,
            pltpu.SemaphoreType.DMA,
        ],
    )(seq, basis_flat)
    return out2.reshape(ROWS, COLS, K)


def kernel(sequence_indices, basis):
    return _su2_lookup(sequence_indices.astype(jnp.int32), basis.reshape(-1))
